# Initial kernel scaffold; baseline (speedup 1.0000x reference)
import jax
import jax.numpy as jnp
from jax.experimental import pallas as pl


def kernel(x, m1, b1big, w2p, b2p, wfc_flat, bfcp):
    raise NotImplementedError("write your pallas kernel here")


# speedup vs baseline: 1.4636x; 1.4636x over previous
import jax
import jax.numpy as jnp
from jax.experimental import pallas as pl
from jax.experimental.pallas import tpu as pltpu

_CP = 128
_K = 5
_H0, _W0 = 28, 28
_OH1, _OW1 = 24, 24
_PW1 = 12
_OH2, _OW2 = 8, 8
_PH2, _PW2 = 4, 4
_TB = 16


def _fused_kernel(x_ref, m1_ref, b1_ref, w2_ref, b2_ref, wfc_ref, bfc_ref,
                  o_ref):
    tb = x_ref.shape[0]
    x = x_ref[...]

    xk = jnp.concatenate([x[:, kh:kh + _OH1, :] for kh in range(_K)], axis=2)
    xk = xk.reshape(tb * _OH1, _K * _W0)
    acc = jnp.dot(xk, m1_ref[...], preferred_element_type=jnp.float32)
    h = jnp.maximum(acc + b1_ref[...], 0.0)

    even = jnp.concatenate([h[:, (2 * j) * _CP:(2 * j + 1) * _CP]
                            for j in range(_PW1)], axis=1)
    odd = jnp.concatenate([h[:, (2 * j + 1) * _CP:(2 * j + 2) * _CP]
                           for j in range(_PW1)], axis=1)
    hw = jnp.maximum(even, odd)

    p1 = jnp.max(hw.reshape(tb, _PW1, 2, _PW1 * _CP), axis=2)
    p1 = p1.astype(jnp.bfloat16)

    cols = []
    for j in range(_PW2):
        accj = jnp.dot(p1[:, 0:_OH2, 2 * j * _CP:(2 * j + 6) * _CP]
                       .reshape(tb * _OH2, 6 * _CP),
                       w2_ref[0], preferred_element_type=jnp.float32)
        for kh in range(1, _K):
            lhs = p1[:, kh:kh + _OH2, 2 * j * _CP:(2 * j + 6) * _CP]
            lhs = lhs.reshape(tb * _OH2, 6 * _CP)
            accj = accj + jnp.dot(lhs, w2_ref[kh],
                                  preferred_element_type=jnp.float32)
        accj = jnp.maximum(accj + b2_ref[...], 0.0)
        cols.append(jnp.maximum(accj[:, :_CP], accj[:, _CP:]))
    hw2 = jnp.concatenate(cols, axis=1)

    p2 = jnp.max(hw2.reshape(tb, _PH2, 2, _PW2 * _CP), axis=2)

    p2w = jnp.concatenate([p2[:, t, :] for t in range(_PH2)], axis=1)
    logits = jnp.dot(p2w.astype(jnp.bfloat16), wfc_ref[...],
                     preferred_element_type=jnp.float32) + bfc_ref[...]
    o_ref[...] = logits


def _pack_w2_banded(w2p):
    w2r = w2p.reshape(_K, _K, _CP, _CP)
    zero = jnp.zeros((_K, _CP, _CP), w2p.dtype)
    rows = []
    for wpp in range(6):
        halves = []
        for opp in range(2):
            kw = wpp - opp
            halves.append(w2r[:, kw] if 0 <= kw < _K else zero)
        rows.append(jnp.concatenate(halves, axis=2))
    return jnp.concatenate(rows, axis=1)


def kernel(x, m1, b1big, w2p, b2p, wfc_flat, bfcp):
    B = x.shape[0]
    xs = x.reshape(B, _H0, _W0).astype(jnp.bfloat16)
    tb = _TB if B >= _TB else B
    Bp = ((B + tb - 1) // tb) * tb
    if Bp != B:
        xs = jnp.pad(xs, ((0, Bp - B), (0, 0), (0, 0)))

    m1b = m1.reshape(_K * _W0, _OW1 * _CP).astype(jnp.bfloat16)
    w2b = _pack_w2_banded(w2p).astype(jnp.bfloat16)
    b2w = jnp.concatenate([b2p, b2p], axis=1)
    wfcb = wfc_flat.astype(jnp.bfloat16)

    grid = (Bp // tb,)
    out = pl.pallas_call(
        _fused_kernel,
        out_shape=jax.ShapeDtypeStruct((Bp, _CP), jnp.float32),
        grid=grid,
        in_specs=[
            pl.BlockSpec((tb, _H0, _W0), lambda i: (i, 0, 0)),
            pl.BlockSpec((_K * _W0, _OW1 * _CP), lambda i: (0, 0)),
            pl.BlockSpec((1, _OW1 * _CP), lambda i: (0, 0)),
            pl.BlockSpec((_K, 6 * _CP, 2 * _CP), lambda i: (0, 0, 0)),
            pl.BlockSpec((1, 2 * _CP), lambda i: (0, 0)),
            pl.BlockSpec((_PH2 * _PW2 * _CP, _CP), lambda i: (0, 0)),
            pl.BlockSpec((1, _CP), lambda i: (0, 0)),
        ],
        out_specs=pl.BlockSpec((tb, _CP), lambda i: (i, 0)),
        compiler_params=pltpu.CompilerParams(
            dimension_semantics=("parallel",),
            vmem_limit_bytes=64 * 1024 * 1024),
    )(xs, m1b, b1big, w2b, b2w, wfcb, bfcp)

    return out[:B, :10]


# speedup vs baseline: 2.2121x; 1.5114x over previous
import jax
import jax.numpy as jnp
from jax.experimental import pallas as pl
from jax.experimental.pallas import tpu as pltpu

_CP = 128
_K = 5
_H0, _W0 = 28, 28
_OH1, _OW1 = 24, 24
_PH1, _PW1 = 12, 12
_OH2, _OW2 = 8, 8
_PH2, _PW2 = 4, 4
_TB = 16
_N1 = _OW1 * _CP
_KC1 = 6 * _W0


def _fused_kernel(xe_ref, xo_ref, m1_ref, b1_ref, w2_ref, b2_ref, wfc_ref,
                  bfc_ref, o_ref):
    tb = xe_ref.shape[0]
    xe = xe_ref[...]
    xo = xo_ref[...]

    xk = jnp.concatenate(
        [xe[:, 0:_PH1, :], xo[:, 0:_PH1, :],
         xe[:, 1:_PH1 + 1, :], xo[:, 1:_PH1 + 1, :],
         xe[:, 2:_PH1 + 2, :], xo[:, 2:_PH1 + 2, :]], axis=2)
    xk = xk.reshape(tb * _PH1, _KC1)
    acc = jnp.dot(xk, m1_ref[...], preferred_element_type=jnp.float32)
    h = jnp.maximum(acc + b1_ref[...], 0.0)

    even = jnp.concatenate(
        [h[:, p * _N1 + (2 * j) * _CP:p * _N1 + (2 * j + 1) * _CP]
         for p in range(2) for j in range(_PW1)], axis=1)
    odd = jnp.concatenate(
        [h[:, p * _N1 + (2 * j + 1) * _CP:p * _N1 + (2 * j + 2) * _CP]
         for p in range(2) for j in range(_PW1)], axis=1)
    hw = jnp.maximum(even, odd)

    half = _PW1 * _CP
    p1 = jnp.maximum(hw[:, :half], hw[:, half:]).astype(jnp.bfloat16)
    p1 = p1.reshape(tb, _PH1, half)

    cols = []
    for j in range(_PW2):
        accj = jnp.dot(p1[:, 0:_OH2, 2 * j * _CP:(2 * j + 6) * _CP]
                       .reshape(tb * _OH2, 6 * _CP),
                       w2_ref[0], preferred_element_type=jnp.float32)
        for kh in range(1, _K):
            lhs = p1[:, kh:kh + _OH2, 2 * j * _CP:(2 * j + 6) * _CP]
            lhs = lhs.reshape(tb * _OH2, 6 * _CP)
            accj = accj + jnp.dot(lhs, w2_ref[kh],
                                  preferred_element_type=jnp.float32)
        accj = jnp.maximum(accj + b2_ref[...], 0.0)
        cols.append(jnp.maximum(accj[:, :_CP], accj[:, _CP:]))
    hw2 = jnp.concatenate(cols, axis=1)

    p2 = jnp.max(hw2.reshape(tb, _PH2, 2, _PW2 * _CP), axis=2)

    p2w = jnp.concatenate([p2[:, t, :] for t in range(_PH2)], axis=1)
    logits = jnp.dot(p2w.astype(jnp.bfloat16), wfc_ref[...],
                     preferred_element_type=jnp.float32) + bfc_ref[...]
    o_ref[...] = logits


def _pack_m1_banded(m1):
    zero = jnp.zeros((_W0, _N1), m1.dtype)
    rows = []
    for r in range(6):
        halves = []
        for p in range(2):
            kh = r - p
            halves.append(m1[kh] if 0 <= kh < _K else zero)
        rows.append(jnp.concatenate(halves, axis=1))
    return jnp.concatenate(rows, axis=0)


def _pack_w2_banded(w2p):
    w2r = w2p.reshape(_K, _K, _CP, _CP)
    zero = jnp.zeros((_K, _CP, _CP), w2p.dtype)
    rows = []
    for wpp in range(6):
        halves = []
        for opp in range(2):
            kw = wpp - opp
            halves.append(w2r[:, kw] if 0 <= kw < _K else zero)
        rows.append(jnp.concatenate(halves, axis=2))
    return jnp.concatenate(rows, axis=1)


def kernel(x, m1, b1big, w2p, b2p, wfc_flat, bfcp):
    B = x.shape[0]
    xs = x.reshape(B, _H0, _W0).astype(jnp.bfloat16)
    tb = _TB if B >= _TB else B
    Bp = ((B + tb - 1) // tb) * tb
    if Bp != B:
        xs = jnp.pad(xs, ((0, Bp - B), (0, 0), (0, 0)))
    xe = xs[:, 0::2, :]
    xo = xs[:, 1::2, :]

    m1b = _pack_m1_banded(m1).astype(jnp.bfloat16)
    b1w = jnp.concatenate([b1big, b1big], axis=1)
    w2b = _pack_w2_banded(w2p).astype(jnp.bfloat16)
    b2w = jnp.concatenate([b2p, b2p], axis=1)
    wfcb = wfc_flat.astype(jnp.bfloat16)

    grid = (Bp // tb,)
    out = pl.pallas_call(
        _fused_kernel,
        out_shape=jax.ShapeDtypeStruct((Bp, _CP), jnp.float32),
        grid=grid,
        in_specs=[
            pl.BlockSpec((tb, _H0 // 2, _W0), lambda i: (i, 0, 0)),
            pl.BlockSpec((tb, _H0 // 2, _W0), lambda i: (i, 0, 0)),
            pl.BlockSpec((_KC1, 2 * _N1), lambda i: (0, 0)),
            pl.BlockSpec((1, 2 * _N1), lambda i: (0, 0)),
            pl.BlockSpec((_K, 6 * _CP, 2 * _CP), lambda i: (0, 0, 0)),
            pl.BlockSpec((1, 2 * _CP), lambda i: (0, 0)),
            pl.BlockSpec((_PH2 * _PW2 * _CP, _CP), lambda i: (0, 0)),
            pl.BlockSpec((1, _CP), lambda i: (0, 0)),
        ],
        out_specs=pl.BlockSpec((tb, _CP), lambda i: (i, 0)),
        compiler_params=pltpu.CompilerParams(
            dimension_semantics=("parallel",),
            vmem_limit_bytes=64 * 1024 * 1024),
    )(xe, xo, m1b, b1w, w2b, b2w, wfcb, bfcp)

    return out[:B, :10]


# speedup vs baseline: 2.4095x; 1.0892x over previous
import jax
import jax.numpy as jnp
from jax.experimental import pallas as pl
from jax.experimental.pallas import tpu as pltpu

_CP = 128
_K = 5
_H0, _W0 = 28, 28
_OH1, _OW1 = 24, 24
_PH1, _PW1 = 12, 12
_OH2, _OW2 = 8, 8
_PH2, _PW2 = 4, 4
_TB = 32
_N1 = _OW1 * _CP
_KC1 = 6 * _W0


def _fused_kernel(xe_ref, xo_ref, m1_ref, b1_ref, w2_ref, b2_ref, wfc_ref,
                  bfc_ref, o_ref):
    tb = xe_ref.shape[0]
    xe = xe_ref[...]
    xo = xo_ref[...]

    xk = jnp.concatenate(
        [xe[:, 0:_PH1, :], xo[:, 0:_PH1, :],
         xe[:, 1:_PH1 + 1, :], xo[:, 1:_PH1 + 1, :],
         xe[:, 2:_PH1 + 2, :], xo[:, 2:_PH1 + 2, :]], axis=2)
    xk = xk.reshape(tb * _PH1, _KC1)
    acc = jnp.dot(xk, m1_ref[...], preferred_element_type=jnp.float32)
    h = jnp.maximum(acc + b1_ref[...], 0.0)

    even = jnp.concatenate(
        [h[:, p * _N1 + (2 * j) * _CP:p * _N1 + (2 * j + 1) * _CP]
         for p in range(2) for j in range(_PW1)], axis=1)
    odd = jnp.concatenate(
        [h[:, p * _N1 + (2 * j + 1) * _CP:p * _N1 + (2 * j + 2) * _CP]
         for p in range(2) for j in range(_PW1)], axis=1)
    hw = jnp.maximum(even, odd)

    half = _PW1 * _CP
    p1 = jnp.maximum(hw[:, :half], hw[:, half:]).astype(jnp.bfloat16)
    p1 = p1.reshape(tb, _PH1, half)

    cols = []
    for j in range(_PW2):
        accj = jnp.dot(p1[:, 0:_OH2, 2 * j * _CP:(2 * j + 6) * _CP]
                       .reshape(tb * _OH2, 6 * _CP),
                       w2_ref[0], preferred_element_type=jnp.float32)
        for kh in range(1, _K):
            lhs = p1[:, kh:kh + _OH2, 2 * j * _CP:(2 * j + 6) * _CP]
            lhs = lhs.reshape(tb * _OH2, 6 * _CP)
            accj = accj + jnp.dot(lhs, w2_ref[kh],
                                  preferred_element_type=jnp.float32)
        accj = jnp.maximum(accj + b2_ref[...], 0.0)
        cols.append(jnp.maximum(accj[:, :_CP], accj[:, _CP:]))
    hw2 = jnp.concatenate(cols, axis=1)

    p2 = jnp.max(hw2.reshape(tb, _PH2, 2, _PW2 * _CP), axis=2)

    p2w = jnp.concatenate([p2[:, t, :] for t in range(_PH2)], axis=1)
    logits = jnp.dot(p2w.astype(jnp.bfloat16), wfc_ref[...],
                     preferred_element_type=jnp.float32) + bfc_ref[...]
    o_ref[...] = logits


def _pack_m1_banded(m1):
    zero = jnp.zeros((_W0, _N1), m1.dtype)
    rows = []
    for r in range(6):
        halves = []
        for p in range(2):
            kh = r - p
            halves.append(m1[kh] if 0 <= kh < _K else zero)
        rows.append(jnp.concatenate(halves, axis=1))
    return jnp.concatenate(rows, axis=0)


def _pack_w2_banded(w2p):
    w2r = w2p.reshape(_K, _K, _CP, _CP)
    zero = jnp.zeros((_K, _CP, _CP), w2p.dtype)
    rows = []
    for wpp in range(6):
        halves = []
        for opp in range(2):
            kw = wpp - opp
            halves.append(w2r[:, kw] if 0 <= kw < _K else zero)
        rows.append(jnp.concatenate(halves, axis=2))
    return jnp.concatenate(rows, axis=1)


def kernel(x, m1, b1big, w2p, b2p, wfc_flat, bfcp):
    B = x.shape[0]
    xs = x.reshape(B, _H0, _W0).astype(jnp.bfloat16)
    tb = _TB if B >= _TB else B
    Bp = ((B + tb - 1) // tb) * tb
    if Bp != B:
        xs = jnp.pad(xs, ((0, Bp - B), (0, 0), (0, 0)))
    xe = xs[:, 0::2, :]
    xo = xs[:, 1::2, :]

    m1b = _pack_m1_banded(m1).astype(jnp.bfloat16)
    b1w = jnp.concatenate([b1big, b1big], axis=1)
    w2b = _pack_w2_banded(w2p).astype(jnp.bfloat16)
    b2w = jnp.concatenate([b2p, b2p], axis=1)
    wfcb = wfc_flat.astype(jnp.bfloat16)

    grid = (Bp // tb,)
    out = pl.pallas_call(
        _fused_kernel,
        out_shape=jax.ShapeDtypeStruct((Bp, _CP), jnp.float32),
        grid=grid,
        in_specs=[
            pl.BlockSpec((tb, _H0 // 2, _W0), lambda i: (i, 0, 0)),
            pl.BlockSpec((tb, _H0 // 2, _W0), lambda i: (i, 0, 0)),
            pl.BlockSpec((_KC1, 2 * _N1), lambda i: (0, 0)),
            pl.BlockSpec((1, 2 * _N1), lambda i: (0, 0)),
            pl.BlockSpec((_K, 6 * _CP, 2 * _CP), lambda i: (0, 0, 0)),
            pl.BlockSpec((1, 2 * _CP), lambda i: (0, 0)),
            pl.BlockSpec((_PH2 * _PW2 * _CP, _CP), lambda i: (0, 0)),
            pl.BlockSpec((1, _CP), lambda i: (0, 0)),
        ],
        out_specs=pl.BlockSpec((tb, _CP), lambda i: (i, 0)),
        compiler_params=pltpu.CompilerParams(
            dimension_semantics=("parallel",),
            vmem_limit_bytes=64 * 1024 * 1024),
    )(xe, xo, m1b, b1w, w2b, b2w, wfcb, bfcp)

    return out[:B, :10]
